# baseline (speedup 1.0000x reference)
import functools

import jax
import jax.numpy as jnp
from jax import lax
from jax.experimental import pallas as pl
from jax.experimental.pallas import tpu as pltpu
from jax.experimental.pallas import tpu_sc as plsc

N = 10000
NPAD = 10240
E = 320000
F_IN = 128
H = 16
C = 40
CP = 48

NC = 2
NS = 16
NW = NC * NS
EPW = E // NW
CB = 80
CHUNKS = EPW // CB
ZONE = NPAD // NS

_mesh = plsc.VectorSubcoreMesh(core_axis_name="c", subcore_axis_name="s")


@functools.partial(
    pl.kernel,
    out_type=jax.ShapeDtypeStruct((NC, NPAD), jnp.float32),
    mesh=_mesh,
    scratch_types=[
        pltpu.VMEM((CHUNKS, CB), jnp.int32),
        pltpu.VMEM((CHUNKS, CB), jnp.float32),
        pltpu.VMEM_SHARED((NPAD,), jnp.float32),
        pltpu.SemaphoreType.DMA,
    ],
)
def _sc_deg(dst_hbm, ew_hbm, zero_hbm, out_hbm, dstv, ewv, acc, sem):
    c = lax.axis_index("c")
    s = lax.axis_index("s")
    wid = s * NC + c
    pltpu.sync_copy(dst_hbm.at[wid], dstv)
    pltpu.sync_copy(ew_hbm.at[wid], ewv)
    pltpu.sync_copy(zero_hbm.at[pl.ds(s * ZONE, ZONE)],
                    acc.at[pl.ds(s * ZONE, ZONE)])
    plsc.subcore_barrier()

    for burst in range(CHUNKS // 25):
        descs = [
            pltpu.async_copy(ewv.at[25 * burst + j],
                             acc.at[dstv.at[25 * burst + j]], sem, add=True)
            for j in range(25)
        ]
        for d in descs:
            d.wait()

    plsc.subcore_barrier()
    pltpu.sync_copy(acc.at[pl.ds(s * ZONE, ZONE)],
                    out_hbm.at[c, pl.ds(s * ZONE, ZONE)])


def _make_edge_pass(D, K, NBUF=2, table_in_spmem=False):
    SUPE = K * CB
    NSUP = CHUNKS // K

    @functools.partial(
        pl.kernel,
        out_type=jax.ShapeDtypeStruct((NC, NPAD, D), jnp.float32),
        mesh=_mesh,
        scratch_types=[
            pltpu.VMEM((CHUNKS, CB), jnp.int32),
            pltpu.VMEM((CHUNKS, CB), jnp.int32),
            pltpu.VMEM((NSUP, SUPE), jnp.float32),
        ] + [pltpu.VMEM((SUPE, D), jnp.float32)] * NBUF + [
            pltpu.VMEM((NPAD,), jnp.float32),
            pltpu.VMEM((1, D), jnp.float32),
            pltpu.VMEM_SHARED((NPAD, D), jnp.float32),
        ] + ([pltpu.VMEM_SHARED((NPAD, D), jnp.float32)]
             if table_in_spmem else []) + [
            pltpu.SemaphoreType.DMA] * (2 * NBUF),
        compiler_params=pltpu.CompilerParams(use_tc_tiling_on_sc=False,
                                             needs_layout_passes=False),
    )
    def edge_pass(src_hbm, dst_hbm, ew_hbm, table_hbm, dis_hbm, bias_hbm,
                  zero_hbm, out_hbm, srcv, dstv, ewv, *rest):
        rows = rest[:NBUF]
        disv, biasv, acc = rest[NBUF:NBUF + 3]
        nfix = NBUF + 3
        if table_in_spmem:
            tsp = rest[nfix]
            nfix += 1
        gsems = rest[nfix:nfix + NBUF]
        ssems = rest[nfix + NBUF:nfix + 2 * NBUF]
        rows0, rows1 = rows[0], rows[1]
        c = lax.axis_index("c")
        s = lax.axis_index("s")
        wid = s * NC + c
        pltpu.sync_copy(src_hbm.at[wid], srcv)
        pltpu.sync_copy(dst_hbm.at[wid], dstv)
        pltpu.sync_copy(ew_hbm.at[wid], ewv)
        pltpu.sync_copy(bias_hbm, biasv)
        pltpu.sync_copy(dis_hbm, disv)
        if table_in_spmem:
            pltpu.sync_copy(table_hbm.at[pl.ds(s * ZONE, ZONE)],
                            tsp.at[pl.ds(s * ZONE, ZONE)])
            gsrc = tsp
        else:
            gsrc = table_hbm

        bvecs = [biasv[0, pl.ds(16 * j, 16)] for j in range(D // 16)]

        if SUPE >= ZONE:
            segs = ((rows0, 0, ZONE),)
        else:
            segs = ((rows0, 0, SUPE), (rows1, SUPE, ZONE - SUPE))

        @pl.when(c == 0)
        def _():
            for buf, n0, nrows in segs:
                pltpu.sync_copy(table_hbm.at[pl.ds(s * ZONE + n0, nrows)],
                                buf.at[pl.ds(0, nrows)])

                def init_group(g, carry, buf=buf, n0=n0):
                    dv = disv[pl.ds(s * ZONE + n0 + g * 16, 16)]
                    base = g * 16
                    for u in range(16):
                        q = dv[u] * dv[u]
                        for j in range(D // 16):
                            buf[base + u, pl.ds(16 * j, 16)] = (
                                buf[base + u, pl.ds(16 * j, 16)] * q
                                + bvecs[j])
                    return carry

                lax.fori_loop(0, nrows // 16, init_group, 0)
                pltpu.sync_copy(buf.at[pl.ds(0, nrows)],
                                acc.at[pl.ds(s * ZONE + n0, nrows)])

        @pl.when(c != 0)
        def _():
            pltpu.sync_copy(zero_hbm.at[pl.ds(s * ZONE, ZONE)],
                            acc.at[pl.ds(s * ZONE, ZONE)])

        plsc.subcore_barrier()

        bufs = tuple(zip(rows, gsems, ssems))

        def issue_gathers(sup, b):
            rb, gs, _ = bufs[b]
            for j in range(K):
                pltpu.async_copy(gsrc.at[srcv.at[sup * K + j]],
                                 rb.at[pl.ds(j * CB, CB)], gs)

        def issue_scatters(sup, b):
            rb, _, ss = bufs[b]
            for j in range(K):
                pltpu.async_copy(rb.at[pl.ds(j * CB, CB)],
                                 acc.at[dstv.at[sup * K + j]], ss, add=True)

        def wait_gathers(b):
            rb, gs, _ = bufs[b]
            for j in range(K):
                pltpu.make_async_copy(gsrc.at[srcv.at[j]],
                                      rb.at[pl.ds(j * CB, CB)], gs).wait()

        def wait_scatters(b):
            rb, _, ss = bufs[b]
            for j in range(K):
                pltpu.make_async_copy(rb.at[pl.ds(j * CB, CB)],
                                      acc.at[dstv.at[j]], ss).wait()

        def scale(sup, b):
            rb = bufs[b][0]

            def group(g, carry):
                row = sup * K + g // 5
                col = (g % 5) * 16
                sg = plsc.load_gather(disv, [srcv[row, pl.ds(col, 16)]])
                dg = plsc.load_gather(disv, [dstv[row, pl.ds(col, 16)]])
                wn = ewv[sup, pl.ds(g * 16, 16)] * sg * dg
                base = g * 16
                for u in range(16):
                    w = wn[u]
                    for j in range(D // 16):
                        rb[base + u, pl.ds(j * 16, 16)] = (
                            rb[base + u, pl.ds(j * 16, 16)] * w)
                return carry

            lax.fori_loop(0, SUPE // 16, group, 0)

        if NBUF == 2:
            issue_gathers(0, 0)
            wait_gathers(0)
            issue_gathers(1, 1)
            scale(0, 0)
            issue_scatters(0, 0)

            def pair(i, carry):
                s0 = 2 * i + 1
                wait_gathers(1)
                wait_scatters(0)
                issue_gathers(s0 + 1, 0)
                scale(s0, 1)
                issue_scatters(s0, 1)
                wait_gathers(0)
                wait_scatters(1)
                issue_gathers(s0 + 2, 1)
                scale(s0 + 1, 0)
                issue_scatters(s0 + 1, 0)
                return carry

            lax.fori_loop(0, (NSUP - 3) // 2, pair, 0)

            wait_gathers(1)
            wait_scatters(0)
            issue_gathers(NSUP - 1, 0)
            scale(NSUP - 2, 1)
            issue_scatters(NSUP - 2, 1)
            wait_gathers(0)
            wait_scatters(1)
            scale(NSUP - 1, 0)
            issue_scatters(NSUP - 1, 0)
            wait_scatters(0)
        else:
            def step(s, b, issue_next=True, wait_prior_scatter=True):
                wait_gathers(b)
                if issue_next:
                    nb = (b + 2) % 3
                    if wait_prior_scatter:
                        wait_scatters(nb)
                    issue_gathers(s + 2, nb)
                scale(s, b)
                issue_scatters(s, b)

            issue_gathers(0, 0)
            issue_gathers(1, 1)
            step(0, 0, wait_prior_scatter=False)
            step(1, 1)

            def triple(i, carry):
                s0 = 3 * i + 2
                step(s0, 2)
                step(s0 + 1, 0)
                step(s0 + 2, 1)
                return carry

            lax.fori_loop(0, (NSUP - 4) // 3, triple, 0)

            step(NSUP - 2, (NSUP - 2) % 3, issue_next=False)
            step(NSUP - 1, (NSUP - 1) % 3, issue_next=False)
            wait_scatters((NSUP - 3) % 3)
            wait_scatters((NSUP - 2) % 3)
            wait_scatters((NSUP - 1) % 3)

        plsc.subcore_barrier()
        pltpu.sync_copy(acc.at[pl.ds(s * ZONE, ZONE)],
                        out_hbm.at[c, pl.ds(s * ZONE, ZONE)])

    return edge_pass


K16 = 25
K48 = 5
_sc_edge16 = _make_edge_pass(H, K16, table_in_spmem=True)
_sc_edge48 = _make_edge_pass(CP, K48, NBUF=3)


def _tc1_body(x_ref, w1_ref, degf_ref, xwn_ref, disf_ref):
    xp = jnp.concatenate(
        [x_ref[...], jnp.zeros((NPAD - N, F_IN), jnp.float32)], axis=0)
    xwn_ref[...] = jnp.dot(xp, w1_ref[...],
                           preferred_element_type=jnp.float32)
    deg = degf_ref[0] + degf_ref[1] + 1.0
    disf_ref[...] = jnp.where(deg > 0, lax.rsqrt(deg), 0.0)


def _tc1(x, W1, degF):
    return pl.pallas_call(
        _tc1_body,
        out_shape=(jax.ShapeDtypeStruct((NPAD, H), jnp.float32),
                   jax.ShapeDtypeStruct((NPAD // 128, 128), jnp.float32)),
    )(x, W1, degF)


def _tc2_body(af_ref, w2_ref, hwf_ref):
    w2 = w2_ref[...]
    w2bd = jnp.concatenate([
        jnp.concatenate([jnp.zeros((H, p * CP), jnp.float32), w2,
                         jnp.zeros((H, (7 - p) * CP), jnp.float32)], axis=1)
        if 0 < p < 7 else
        (jnp.concatenate([w2, jnp.zeros((H, 7 * CP), jnp.float32)], axis=1)
         if p == 0 else
         jnp.concatenate([jnp.zeros((H, 7 * CP), jnp.float32), w2], axis=1))
        for p in range(8)
    ], axis=0)
    h = jnp.maximum(af_ref[0] + af_ref[1], 0.0)
    hwf_ref[...] = jnp.dot(h, w2bd, preferred_element_type=jnp.float32)


def _tc2(af, W2p):
    return pl.pallas_call(
        _tc2_body,
        out_shape=jax.ShapeDtypeStruct((NPAD // 8, 8 * CP), jnp.float32),
    )(af, W2p)


def _tc3a_body(bf_ref, zf_ref):
    zf_ref[...] = bf_ref[0] + bf_ref[1]


def _tc3a(bf):
    return pl.pallas_call(
        _tc3a_body,
        out_shape=jax.ShapeDtypeStruct((NPAD * CP // 128, 128), jnp.float32),
    )(bf)


def _tc3b_body(z_ref, out_ref):
    z = z_ref[...]
    m = jnp.max(z, axis=1, keepdims=True)
    e = jnp.exp(z - m)
    lsm = z - m - jnp.log(jnp.sum(e, axis=1, keepdims=True))
    out_ref[...] = lsm[:N, :C]


def _tc3b(zN):
    return pl.pallas_call(
        _tc3b_body,
        out_shape=jax.ShapeDtypeStruct((N, C), jnp.float32),
    )(zN)


def kernel(x, edge_index, edge_attr, W1, b1, W2, b2):
    src3 = edge_index[0].reshape(NW, CHUNKS, CB)
    dst3 = edge_index[1].reshape(NW, CHUNKS, CB)
    ew3 = edge_attr.reshape(NW, CHUNKS, CB)
    ewS16 = edge_attr.reshape(NW, CHUNKS // K16, K16 * CB)
    ewS48 = edge_attr.reshape(NW, CHUNKS // K48, K48 * CB)
    W2p = jnp.pad(W2, ((0, 0), (0, CP - C)))
    b1r = b1.reshape(1, H)
    b2r = jnp.concatenate(
        [b2, jnp.full((CP - C,), -1e30, jnp.float32)]).reshape(1, CP)
    zero1 = jnp.zeros((NPAD,), jnp.float32)
    zero16 = jnp.zeros((NPAD, H), jnp.float32)
    zero48 = jnp.zeros((NPAD, CP), jnp.float32)

    degp = _sc_deg(dst3, ew3, zero1)
    degF = degp.reshape(NC, NPAD // 128, 128)
    xwN, disF = _tc1(x, W1, degF)
    disN = disF.reshape(NPAD)
    A = _sc_edge16(src3, dst3, ewS16, xwN, disN, b1r, zero16)
    af = A.reshape(NC, NPAD * H // 128, 128)
    hwF = _tc2(af, W2p)
    hwN = hwF.reshape(NPAD, CP)
    B = _sc_edge48(src3, dst3, ewS48, hwN, disN, b2r, zero48)
    bf = B.reshape(NC, NPAD * CP // 128, 128)
    zf = _tc3a(bf)
    zN = zf.reshape(NPAD, CP)
    return _tc3b(zN)


# speedup vs baseline: 1.0072x; 1.0072x over previous
import functools

import jax
import jax.numpy as jnp
from jax import lax
from jax.experimental import pallas as pl
from jax.experimental.pallas import tpu as pltpu
from jax.experimental.pallas import tpu_sc as plsc

N = 10000
NPAD = 10240
E = 320000
F_IN = 128
H = 16
C = 40
CP = 48

NC = 2
NS = 16
NW = NC * NS
EPW = E // NW
CB = 80
CHUNKS = EPW // CB
ZONE = NPAD // NS

_mesh = plsc.VectorSubcoreMesh(core_axis_name="c", subcore_axis_name="s")


@functools.partial(
    pl.kernel,
    out_type=jax.ShapeDtypeStruct((NC, NPAD), jnp.float32),
    mesh=_mesh,
    scratch_types=[
        pltpu.VMEM((CHUNKS, CB), jnp.int32),
        pltpu.VMEM((CHUNKS, CB), jnp.float32),
        pltpu.VMEM_SHARED((NPAD,), jnp.float32),
        pltpu.SemaphoreType.DMA,
    ],
)
def _sc_deg(dst_hbm, ew_hbm, zero_hbm, out_hbm, dstv, ewv, acc, sem):
    c = lax.axis_index("c")
    s = lax.axis_index("s")
    wid = s * NC + c
    pltpu.sync_copy(dst_hbm.at[wid], dstv)
    pltpu.sync_copy(ew_hbm.at[wid], ewv)
    pltpu.sync_copy(zero_hbm.at[pl.ds(s * ZONE, ZONE)],
                    acc.at[pl.ds(s * ZONE, ZONE)])
    plsc.subcore_barrier()

    for burst in range(CHUNKS // 25):
        descs = [
            pltpu.async_copy(ewv.at[25 * burst + j],
                             acc.at[dstv.at[25 * burst + j]], sem, add=True)
            for j in range(25)
        ]
        for d in descs:
            d.wait()

    plsc.subcore_barrier()
    pltpu.sync_copy(acc.at[pl.ds(s * ZONE, ZONE)],
                    out_hbm.at[c, pl.ds(s * ZONE, ZONE)])


def _make_edge_pass(D, K, NBUF=2):
    SUPE = K * CB
    NSUP = CHUNKS // K

    @functools.partial(
        pl.kernel,
        out_type=jax.ShapeDtypeStruct((NC, NPAD, D), jnp.float32),
        mesh=_mesh,
        scratch_types=[
            pltpu.VMEM((CHUNKS, CB), jnp.int32),
            pltpu.VMEM((CHUNKS, CB), jnp.int32),
            pltpu.VMEM((NSUP, SUPE), jnp.float32),
        ] + [pltpu.VMEM((SUPE, D), jnp.float32)] * NBUF + [
            pltpu.VMEM((NPAD,), jnp.float32),
            pltpu.VMEM((1, D), jnp.float32),
            pltpu.VMEM_SHARED((NPAD, D), jnp.float32),
        ] + [pltpu.SemaphoreType.DMA] * (2 * NBUF),
        compiler_params=pltpu.CompilerParams(use_tc_tiling_on_sc=False,
                                             needs_layout_passes=False),
    )
    def edge_pass(src_hbm, dst_hbm, ew_hbm, table_hbm, dis_hbm, bias_hbm,
                  zero_hbm, out_hbm, srcv, dstv, ewv, *rest):
        rows = rest[:NBUF]
        disv, biasv, acc = rest[NBUF:NBUF + 3]
        gsems = rest[NBUF + 3:2 * NBUF + 3]
        ssems = rest[2 * NBUF + 3:3 * NBUF + 3]
        rows0, rows1 = rows[0], rows[1]
        c = lax.axis_index("c")
        s = lax.axis_index("s")
        wid = s * NC + c
        pltpu.sync_copy(src_hbm.at[wid], srcv)
        pltpu.sync_copy(dst_hbm.at[wid], dstv)
        pltpu.sync_copy(ew_hbm.at[wid], ewv)
        pltpu.sync_copy(bias_hbm, biasv)
        pltpu.sync_copy(dis_hbm, disv)

        bvecs = [biasv[0, pl.ds(16 * j, 16)] for j in range(D // 16)]

        if SUPE >= ZONE:
            segs = ((rows0, 0, ZONE),)
        else:
            segs = ((rows0, 0, SUPE), (rows1, SUPE, ZONE - SUPE))

        @pl.when(c == 0)
        def _():
            for buf, n0, nrows in segs:
                pltpu.sync_copy(table_hbm.at[pl.ds(s * ZONE + n0, nrows)],
                                buf.at[pl.ds(0, nrows)])

                def init_group(g, carry, buf=buf, n0=n0):
                    dv = disv[pl.ds(s * ZONE + n0 + g * 16, 16)]
                    base = g * 16
                    for u in range(16):
                        q = dv[u] * dv[u]
                        for j in range(D // 16):
                            buf[base + u, pl.ds(16 * j, 16)] = (
                                buf[base + u, pl.ds(16 * j, 16)] * q
                                + bvecs[j])
                    return carry

                lax.fori_loop(0, nrows // 16, init_group, 0)
                pltpu.sync_copy(buf.at[pl.ds(0, nrows)],
                                acc.at[pl.ds(s * ZONE + n0, nrows)])

        @pl.when(c != 0)
        def _():
            pltpu.sync_copy(zero_hbm.at[pl.ds(s * ZONE, ZONE)],
                            acc.at[pl.ds(s * ZONE, ZONE)])

        plsc.subcore_barrier()

        bufs = tuple(zip(rows, gsems, ssems))

        def issue_gathers(sup, b):
            rb, gs, _ = bufs[b]
            for j in range(K):
                pltpu.async_copy(table_hbm.at[srcv.at[sup * K + j]],
                                 rb.at[pl.ds(j * CB, CB)], gs)

        def issue_scatters(sup, b):
            rb, _, ss = bufs[b]
            for j in range(K):
                pltpu.async_copy(rb.at[pl.ds(j * CB, CB)],
                                 acc.at[dstv.at[sup * K + j]], ss, add=True)

        def wait_gathers(b):
            rb, gs, _ = bufs[b]
            for j in range(K):
                pltpu.make_async_copy(table_hbm.at[srcv.at[j]],
                                      rb.at[pl.ds(j * CB, CB)], gs).wait()

        def wait_scatters(b):
            rb, _, ss = bufs[b]
            for j in range(K):
                pltpu.make_async_copy(rb.at[pl.ds(j * CB, CB)],
                                      acc.at[dstv.at[j]], ss).wait()

        def scale(sup, b):
            rb = bufs[b][0]

            def group(g, carry):
                row = sup * K + g // 5
                col = (g % 5) * 16
                sg = plsc.load_gather(disv, [srcv[row, pl.ds(col, 16)]])
                dg = plsc.load_gather(disv, [dstv[row, pl.ds(col, 16)]])
                wn = ewv[sup, pl.ds(g * 16, 16)] * sg * dg
                base = g * 16
                for u in range(16):
                    w = wn[u]
                    for j in range(D // 16):
                        rb[base + u, pl.ds(j * 16, 16)] = (
                            rb[base + u, pl.ds(j * 16, 16)] * w)
                return carry

            lax.fori_loop(0, SUPE // 16, group, 0)

        if NBUF == 2:
            issue_gathers(0, 0)
            wait_gathers(0)
            issue_gathers(1, 1)
            scale(0, 0)
            issue_scatters(0, 0)

            def pair(i, carry):
                s0 = 2 * i + 1
                wait_gathers(1)
                wait_scatters(0)
                issue_gathers(s0 + 1, 0)
                scale(s0, 1)
                issue_scatters(s0, 1)
                wait_gathers(0)
                wait_scatters(1)
                issue_gathers(s0 + 2, 1)
                scale(s0 + 1, 0)
                issue_scatters(s0 + 1, 0)
                return carry

            lax.fori_loop(0, (NSUP - 3) // 2, pair, 0)

            wait_gathers(1)
            wait_scatters(0)
            issue_gathers(NSUP - 1, 0)
            scale(NSUP - 2, 1)
            issue_scatters(NSUP - 2, 1)
            wait_gathers(0)
            wait_scatters(1)
            scale(NSUP - 1, 0)
            issue_scatters(NSUP - 1, 0)
            wait_scatters(0)
        else:
            def step(s, b, issue_next=True, wait_prior_scatter=True):
                wait_gathers(b)
                if issue_next:
                    nb = (b + 2) % 3
                    if wait_prior_scatter:
                        wait_scatters(nb)
                    issue_gathers(s + 2, nb)
                scale(s, b)
                issue_scatters(s, b)

            issue_gathers(0, 0)
            issue_gathers(1, 1)
            step(0, 0, wait_prior_scatter=False)
            step(1, 1)

            def triple(i, carry):
                s0 = 3 * i + 2
                step(s0, 2)
                step(s0 + 1, 0)
                step(s0 + 2, 1)
                return carry

            lax.fori_loop(0, (NSUP - 4) // 3, triple, 0)

            step(NSUP - 2, (NSUP - 2) % 3, issue_next=False)
            step(NSUP - 1, (NSUP - 1) % 3, issue_next=False)
            wait_scatters((NSUP - 3) % 3)
            wait_scatters((NSUP - 2) % 3)
            wait_scatters((NSUP - 1) % 3)

        plsc.subcore_barrier()
        pltpu.sync_copy(acc.at[pl.ds(s * ZONE, ZONE)],
                        out_hbm.at[c, pl.ds(s * ZONE, ZONE)])

    return edge_pass


K16 = 25
K48 = 5
_sc_edge16 = _make_edge_pass(H, K16)
_sc_edge48 = _make_edge_pass(CP, K48, NBUF=3)


def _tc1_body(x_ref, w1_ref, degf_ref, xwn_ref, disf_ref):
    xp = jnp.concatenate(
        [x_ref[...], jnp.zeros((NPAD - N, F_IN), jnp.float32)], axis=0)
    xwn_ref[...] = jnp.dot(xp, w1_ref[...],
                           preferred_element_type=jnp.float32)
    deg = degf_ref[0] + degf_ref[1] + 1.0
    disf_ref[...] = jnp.where(deg > 0, lax.rsqrt(deg), 0.0)


def _tc1(x, W1, degF):
    return pl.pallas_call(
        _tc1_body,
        out_shape=(jax.ShapeDtypeStruct((NPAD, H), jnp.float32),
                   jax.ShapeDtypeStruct((NPAD // 128, 128), jnp.float32)),
    )(x, W1, degF)


def _tc2_body(af_ref, w2_ref, hwf_ref):
    w2 = w2_ref[...]
    w2bd = jnp.concatenate([
        jnp.concatenate([jnp.zeros((H, p * CP), jnp.float32), w2,
                         jnp.zeros((H, (7 - p) * CP), jnp.float32)], axis=1)
        if 0 < p < 7 else
        (jnp.concatenate([w2, jnp.zeros((H, 7 * CP), jnp.float32)], axis=1)
         if p == 0 else
         jnp.concatenate([jnp.zeros((H, 7 * CP), jnp.float32), w2], axis=1))
        for p in range(8)
    ], axis=0)
    h = jnp.maximum(af_ref[0] + af_ref[1], 0.0)
    hwf_ref[...] = jnp.dot(h, w2bd, preferred_element_type=jnp.float32)


def _tc2(af, W2p):
    return pl.pallas_call(
        _tc2_body,
        out_shape=jax.ShapeDtypeStruct((NPAD // 8, 8 * CP), jnp.float32),
    )(af, W2p)


def _tc3a_body(bf_ref, zf_ref):
    zf_ref[...] = bf_ref[0] + bf_ref[1]


def _tc3a(bf):
    return pl.pallas_call(
        _tc3a_body,
        out_shape=jax.ShapeDtypeStruct((NPAD * CP // 128, 128), jnp.float32),
    )(bf)


def _tc3b_body(z_ref, out_ref):
    z = z_ref[...]
    m = jnp.max(z, axis=1, keepdims=True)
    e = jnp.exp(z - m)
    lsm = z - m - jnp.log(jnp.sum(e, axis=1, keepdims=True))
    out_ref[...] = lsm[:N, :C]


def _tc3b(zN):
    return pl.pallas_call(
        _tc3b_body,
        out_shape=jax.ShapeDtypeStruct((N, C), jnp.float32),
    )(zN)


def kernel(x, edge_index, edge_attr, W1, b1, W2, b2):
    src3 = edge_index[0].reshape(NW, CHUNKS, CB)
    dst3 = edge_index[1].reshape(NW, CHUNKS, CB)
    ew3 = edge_attr.reshape(NW, CHUNKS, CB)
    ewS16 = edge_attr.reshape(NW, CHUNKS // K16, K16 * CB)
    ewS48 = edge_attr.reshape(NW, CHUNKS // K48, K48 * CB)
    W2p = jnp.pad(W2, ((0, 0), (0, CP - C)))
    b1r = b1.reshape(1, H)
    b2r = jnp.concatenate(
        [b2, jnp.full((CP - C,), -1e30, jnp.float32)]).reshape(1, CP)
    zero1 = jnp.zeros((NPAD,), jnp.float32)
    zero16 = jnp.zeros((NPAD, H), jnp.float32)
    zero48 = jnp.zeros((NPAD, CP), jnp.float32)

    degp = _sc_deg(dst3, ew3, zero1)
    degF = degp.reshape(NC, NPAD // 128, 128)
    xwN, disF = _tc1(x, W1, degF)
    disN = disF.reshape(NPAD)
    A = _sc_edge16(src3, dst3, ewS16, xwN, disN, b1r, zero16)
    af = A.reshape(NC, NPAD * H // 128, 128)
    hwF = _tc2(af, W2p)
    hwN = hwF.reshape(NPAD, CP)
    B = _sc_edge48(src3, dst3, ewS48, hwN, disN, b2r, zero48)
    bf = B.reshape(NC, NPAD * CP // 128, 128)
    zf = _tc3a(bf)
    zN = zf.reshape(NPAD, CP)
    return _tc3b(zN)
